# baseline (speedup 1.0000x reference)
import functools

import jax
import jax.numpy as jnp
from jax import lax
from jax.experimental import pallas as pl
from jax.experimental.pallas import tpu as pltpu
from jax.experimental.pallas import tpu_sc as plsc

N_NODES = 10000
N_EDGES = 320000
IN_F = 16
OUT_F = 16
N_RBF = 16
CUTOFF = 5.0
N_HARM = 9
D = IN_F * N_HARM
DA = 128
DB = D - DA

NK = 5
CE = N_EDGES // NK
NW = 32
CHUNK = 80
PER_W = CE // NW
N_CHUNKS = PER_W // CHUNK

ROWS_PER_TILE = N_NODES // 16
WB = 25
N_WB = ROWS_PER_TILE // WB

RPS = 624
WBM = 208
NWBM = RPS // WBM
REM0 = N_NODES - 16 * RPS

EB = 2560

_SC_LINEAR = pltpu.CompilerParams(use_tc_tiling_on_sc=False)
_SC_TILED = pltpu.CompilerParams(use_tc_tiling_on_sc=True)


def _widx():
    return lax.axis_index("s") * 2 + lax.axis_index("c")


def _gather_body(x_hbm, src_hbm, out_hbm, idx_all, rows, gsem, wsem):
    base = _widx() * PER_W
    pltpu.sync_copy(src_hbm.at[pl.ds(base, PER_W)], idx_all)

    def gstart(c, b):
        isl = idx_all.at[pl.ds(c * CHUNK, CHUNK)]
        pltpu.async_copy(x_hbm.at[isl], rows.at[b], gsem)

    def gwait(b):
        isl = idx_all.at[pl.ds(0, CHUNK)]
        pltpu.make_async_copy(x_hbm.at[isl], rows.at[b], gsem).wait()

    def wstart(c, b):
        off = base + c * CHUNK
        pltpu.async_copy(rows.at[b], out_hbm.at[pl.ds(off, CHUNK)], wsem)

    def wwait(b):
        off = base
        pltpu.make_async_copy(
            rows.at[b], out_hbm.at[pl.ds(off, CHUNK)], wsem).wait()

    gstart(0, 0)

    @pl.loop(0, N_CHUNKS - 1, step=2)
    def _pair(j0):
        gwait(0)

        @pl.when(j0 > 0)
        def _():
            wwait(1)
        gstart(j0 + 1, 1)
        wstart(j0, 0)
        gwait(1)
        wwait(0)
        gstart(j0 + 2, 0)
        wstart(j0 + 1, 1)

    gwait(0)
    wwait(1)
    wstart(N_CHUNKS - 1, 0)
    wwait(0)


def _sc_gather_main(x2a, src):
    mesh = plsc.VectorSubcoreMesh(core_axis_name="c", subcore_axis_name="s")

    @functools.partial(
        pl.kernel,
        mesh=mesh,
        out_type=jax.ShapeDtypeStruct((CE, DA), jnp.float32),
        scratch_types=[
            pltpu.VMEM((PER_W,), jnp.int32),
            pltpu.VMEM((2, CHUNK, DA), jnp.float32),
            pltpu.SemaphoreType.DMA,
            pltpu.SemaphoreType.DMA,
        ],
        compiler_params=_SC_TILED,
    )
    def k(xa_hbm, src_hbm, outa_hbm, idx_all, rows, gsem, wsem):
        _gather_body(xa_hbm, src_hbm, outa_hbm, idx_all, rows, gsem, wsem)

    return k(x2a, src)


def _sc_gather_tail(x2b, src):
    mesh = plsc.VectorSubcoreMesh(core_axis_name="c", subcore_axis_name="s")

    @functools.partial(
        pl.kernel,
        mesh=mesh,
        out_type=jax.ShapeDtypeStruct((CE, DB), jnp.float32),
        scratch_types=[
            pltpu.VMEM((PER_W,), jnp.int32),
            pltpu.VMEM((2, CHUNK, DB), jnp.float32),
            pltpu.SemaphoreType.DMA,
            pltpu.SemaphoreType.DMA,
        ],
        compiler_params=_SC_LINEAR,
    )
    def k(xb_hbm, src_hbm, outb_hbm, idx_all, rows, gsem, wsem):
        _gather_body(xb_hbm, src_hbm, outb_hbm, idx_all, rows, gsem, wsem)

    return k(x2b, src)


def _scatter_pipeline(m_hbm, dst_hbm, acc, idx2, rows, sem):
    base = _widx() * PER_W

    def lstart(c, b):
        off = base + c * CHUNK
        pltpu.async_copy(dst_hbm.at[pl.ds(off, CHUNK)], idx2.at[b], sem)
        pltpu.async_copy(m_hbm.at[pl.ds(off, CHUNK)], rows.at[b], sem)

    def lwait(b):
        off = base
        pltpu.make_async_copy(
            dst_hbm.at[pl.ds(off, CHUNK)], idx2.at[b], sem).wait()
        pltpu.make_async_copy(
            m_hbm.at[pl.ds(off, CHUNK)], rows.at[b], sem).wait()

    def sadd(b):
        idxr = idx2.at[b]
        pltpu.sync_copy(rows.at[b], acc.at[idxr], add=True)

    lstart(0, 0)

    @pl.loop(0, N_CHUNKS - 1, step=2)
    def _pair(j0):
        lwait(0)
        lstart(j0 + 1, 1)
        sadd(0)
        lwait(1)
        lstart(j0 + 2, 0)
        sadd(1)

    lwait(0)
    sadd(0)


def _sc_scatter_main(msgA, dst, zrows):
    mesh = plsc.VectorSubcoreMesh(core_axis_name="c", subcore_axis_name="s")

    @functools.partial(
        pl.kernel,
        mesh=mesh,
        out_type=(jax.ShapeDtypeStruct((N_NODES, DA), jnp.float32),
                  jax.ShapeDtypeStruct((N_NODES, DA), jnp.float32)),
        scratch_types=[
            pltpu.VMEM((2, CHUNK), jnp.int32),
            pltpu.VMEM((2, CHUNK, DA), jnp.float32),
            pltpu.VMEM((WBM, DA), jnp.float32),
            pltpu.VMEM_SHARED((N_NODES, DA), jnp.float32),
            pltpu.SemaphoreType.DMA,
        ],
        compiler_params=_SC_TILED,
    )
    def k(ma_hbm, dst_hbm, z_hbm, pa0_hbm, pa1_hbm,
          idx2, rows, stage, acc, sem):
        cid = lax.axis_index("c")
        sid = lax.axis_index("s")

        r0 = sid * RPS

        def zcp(kk, carry):
            pltpu.sync_copy(z_hbm, acc.at[pl.ds(r0 + kk * WBM, WBM)])
            return carry

        lax.fori_loop(0, NWBM, zcp, 0)

        @pl.when(sid == 0)
        def _():
            pltpu.sync_copy(z_hbm.at[pl.ds(0, REM0)],
                            acc.at[pl.ds(16 * RPS, REM0)])

        plsc.subcore_barrier()

        _scatter_pipeline(ma_hbm, dst_hbm, acc, idx2, rows, sem)

        plsc.subcore_barrier()

        pa_hbm = [pa0_hbm, pa1_hbm]

        def wb(kk, carry):
            rr = r0 + kk * WBM
            pltpu.sync_copy(acc.at[pl.ds(rr, WBM)], stage)
            for c in range(2):
                @pl.when(cid == c)
                def _():
                    pltpu.sync_copy(stage, pa_hbm[c].at[pl.ds(rr, WBM)])
            return carry

        lax.fori_loop(0, NWBM, wb, 0)

        @pl.when(sid == 0)
        def _():
            rr = 16 * RPS
            pltpu.sync_copy(acc.at[pl.ds(rr, REM0)], stage.at[pl.ds(0, REM0)])
            for c in range(2):
                @pl.when(cid == c)
                def _():
                    pltpu.sync_copy(stage.at[pl.ds(0, REM0)],
                                    pa_hbm[c].at[pl.ds(rr, REM0)])

    return k(msgA, dst, zrows)


def _sc_scatter_tail(msgB, dst):
    mesh = plsc.VectorSubcoreMesh(core_axis_name="c", subcore_axis_name="s")

    @functools.partial(
        pl.kernel,
        mesh=mesh,
        out_type=(jax.ShapeDtypeStruct((N_NODES, DB), jnp.float32),
                  jax.ShapeDtypeStruct((N_NODES, DB), jnp.float32)),
        scratch_types=[
            pltpu.VMEM((2, CHUNK), jnp.int32),
            pltpu.VMEM((2, CHUNK, DB), jnp.float32),
            pltpu.VMEM((WB, DB), jnp.float32),
            pltpu.VMEM_SHARED((N_NODES, DB), jnp.float32),
            pltpu.SemaphoreType.DMA,
        ],
        compiler_params=_SC_LINEAR,
    )
    def k(mb_hbm, dst_hbm, pb0_hbm, pb1_hbm, idx2, rows, stage, acc, sem):
        cid = lax.axis_index("c")
        sid = lax.axis_index("s")

        zeros16 = jnp.zeros((16,), jnp.float32)

        def zrow(i, carry):
            stage[i, :] = zeros16
            return carry

        lax.fori_loop(0, WB, zrow, 0)

        def zcp(kk, carry):
            r0 = sid * ROWS_PER_TILE + kk * WB
            pltpu.sync_copy(stage, acc.at[pl.ds(r0, WB)])
            return carry

        lax.fori_loop(0, N_WB, zcp, 0)
        plsc.subcore_barrier()

        _scatter_pipeline(mb_hbm, dst_hbm, acc, idx2, rows, sem)

        plsc.subcore_barrier()

        pb_hbm = [pb0_hbm, pb1_hbm]

        def wb(kk, carry):
            r0 = sid * ROWS_PER_TILE + kk * WB
            pltpu.sync_copy(acc.at[pl.ds(r0, WB)], stage)
            for c in range(2):
                @pl.when(cid == c)
                def _():
                    pltpu.sync_copy(stage, pb_hbm[c].at[pl.ds(r0, WB)])
            return carry

        lax.fori_loop(0, N_WB, wb, 0)

    return k(msgB, dst)


def _tc_body(evT_ref, xsa_ref, xsb_ref, w_ref, w1t_ref, b1_ref, w2t_ref,
             b2_ref, w3t_ref, b3_ref, outa_ref, outb_ref):
    f32 = jnp.float32
    ev = evT_ref[...]
    r2 = jnp.sum(ev * ev, axis=0, keepdims=True)
    r = jnp.sqrt(r2)
    inv = 1.0 / (r + 1e-8)
    vx = ev[0:1] * inv
    vy = ev[1:2] * inv
    vz = ev[2:3] * inv
    c0 = 0.28209479177387814
    c1 = 0.4886025119029199
    c2a = 1.0925484305920792
    c2b = 0.31539156525252005
    c2c = 0.5462742152960396
    sh9 = jnp.concatenate([
        c0 * jnp.ones_like(vx),
        c1 * vy,
        c1 * vz,
        c1 * vx,
        c2a * vx * vy,
        c2a * vy * vz,
        c2b * (3.0 * vz * vz - 1.0),
        c2a * vx * vz,
        c2c * (vx * vx - vy * vy),
    ], axis=0)

    centers = (CUTOFF / (N_RBF - 1)) * lax.broadcasted_iota(
        jnp.int32, (N_RBF, 1), 0).astype(f32)
    w = w_ref[...]
    rbf = jnp.exp(-((r - centers) ** 2) / (2.0 * w * w))
    cut = 0.5 * (1.0 + jnp.cos(jnp.pi * r / CUTOFF)) * (r < CUTOFF).astype(f32)

    def mm(a, b):
        return jax.lax.dot_general(a, b, (((1,), (0,)), ((), ())),
                                   preferred_element_type=f32)

    def silu(h):
        return h / (1.0 + jnp.exp(-h))

    h1 = silu(mm(w1t_ref[...], rbf) + b1_ref[...])
    h2 = silu(mm(w2t_ref[...], h1) + b2_ref[...])
    rwT = mm(w3t_ref[...], h2) + b3_ref[...]

    xsT = jnp.concatenate([xsa_ref[...].T, xsb_ref[...].T], axis=0)
    sh9c = sh9 * cut
    shtile = jnp.concatenate([sh9c] * IN_F, axis=0)
    m = xsT * shtile

    parts = []
    for h in range(N_HARM):
        acc = None
        for i in range(IN_F):
            mrow = m[i * N_HARM + h:i * N_HARM + h + 1]
            term = rwT[i * OUT_F:(i + 1) * OUT_F] * mrow
            acc = term if acc is None else acc + term
        parts.append(acc)
    msg = jnp.concatenate(parts, axis=0)
    outa_ref[...] = msg[0:DA].T
    outb_ref[...] = msg[DA:D].T


def _tc_compute(evT, xsA, xsB, widths, w1t, b1c, w2t, b2c, w3tp, b3cp):
    nb = CE // EB
    return pl.pallas_call(
        _tc_body,
        grid=(nb,),
        in_specs=[
            pl.BlockSpec((3, EB), lambda i: (0, i)),
            pl.BlockSpec((EB, DA), lambda i: (i, 0)),
            pl.BlockSpec((EB, DB), lambda i: (i, 0)),
            pl.BlockSpec((N_RBF, 1), lambda i: (0, 0)),
            pl.BlockSpec((64, N_RBF), lambda i: (0, 0)),
            pl.BlockSpec((64, 1), lambda i: (0, 0)),
            pl.BlockSpec((64, 64), lambda i: (0, 0)),
            pl.BlockSpec((64, 1), lambda i: (0, 0)),
            pl.BlockSpec((256, 64), lambda i: (0, 0)),
            pl.BlockSpec((256, 1), lambda i: (0, 0)),
        ],
        out_specs=(pl.BlockSpec((EB, DA), lambda i: (i, 0)),
                   pl.BlockSpec((EB, DB), lambda i: (i, 0))),
        out_shape=(jax.ShapeDtypeStruct((CE, DA), jnp.float32),
                   jax.ShapeDtypeStruct((CE, DB), jnp.float32)),
    )(evT, xsA, xsB, widths, w1t, b1c, w2t, b2c, w3tp, b3cp)


def _combine_body(*refs):
    n = (len(refs) - 2) // 2
    pa = refs[:n]
    pb = refs[n:2 * n]
    oa_ref, ob_ref = refs[-2], refs[-1]
    accA = None
    accB = None
    for ra, rb in zip(pa, pb):
        accA = ra[...] if accA is None else accA + ra[...]
        accB = rb[...] if accB is None else accB + rb[...]
    oa_ref[...] = accA
    ob_ref[...] = accB


def _tc_combine(pAs, pBs):
    nb = 5
    rb = N_NODES // nb
    return pl.pallas_call(
        _combine_body,
        grid=(nb,),
        in_specs=([pl.BlockSpec((rb, DA), lambda i: (i, 0)) for _ in pAs]
                  + [pl.BlockSpec((rb, DB), lambda i: (i, 0)) for _ in pBs]),
        out_specs=(pl.BlockSpec((rb, DA), lambda i: (i, 0)),
                   pl.BlockSpec((rb, DB), lambda i: (i, 0))),
        out_shape=(jax.ShapeDtypeStruct((N_NODES, DA), jnp.float32),
                   jax.ShapeDtypeStruct((N_NODES, DB), jnp.float32)),
    )(*pAs, *pBs)


def kernel(x, edge_index, edge_vec, widths, W1, b1, W2, b2, W3, b3):
    x2 = x.reshape(N_NODES, D)
    x2a = x2[:, :DA]
    x2b = x2[:, DA:]
    src = edge_index[0]
    dst = edge_index[1]
    evT = edge_vec.T
    zrows = jnp.zeros((WBM, DA), jnp.float32)

    w1t = W1.T
    w2t = W2.T
    w3p = W3.reshape(64, OUT_F, IN_F).transpose(0, 2, 1).reshape(64, OUT_F * IN_F)
    w3tp = w3p.T
    b3p = b3.reshape(OUT_F, IN_F).T.reshape(OUT_F * IN_F, 1)
    b1c = b1.reshape(64, 1)
    b2c = b2.reshape(64, 1)
    wc = widths.reshape(N_RBF, 1)

    pAs, pBs = [], []
    for k in range(NK):
        lo, hi = k * CE, (k + 1) * CE
        xsA = _sc_gather_main(x2a, src[lo:hi])
        xsB = _sc_gather_tail(x2b, src[lo:hi])
        msgA, msgB = _tc_compute(evT[:, lo:hi], xsA, xsB, wc, w1t, b1c,
                                 w2t, b2c, w3tp, b3p)
        pa0, pa1 = _sc_scatter_main(msgA, dst[lo:hi], zrows)
        pb0, pb1 = _sc_scatter_tail(msgB, dst[lo:hi])
        pAs += [pa0, pa1]
        pBs += [pb0, pb1]

    outA, outB = _tc_combine(pAs, pBs)
    a = outA.reshape(N_NODES, 8, OUT_F)
    b = outB.reshape(N_NODES, 1, OUT_F)
    return jnp.concatenate([a, b], axis=1).transpose(0, 2, 1)


# speedup vs baseline: 1.1729x; 1.1729x over previous
import functools

import jax
import jax.numpy as jnp
from jax import lax
from jax.experimental import pallas as pl
from jax.experimental.pallas import tpu as pltpu
from jax.experimental.pallas import tpu_sc as plsc

N_NODES = 10000
N_EDGES = 320000
IN_F = 16
OUT_F = 16
N_RBF = 16
CUTOFF = 5.0
N_HARM = 9
D = IN_F * N_HARM
DA = 128
DB = D - DA

SCHED = [12800, 89600, 89600, 64000, 38400, 12800, 12800]
NW = 32
CHUNK = 80

ROWS_PER_TILE = N_NODES // 16
WB = 25
N_WB = ROWS_PER_TILE // WB

EB = 2560

_SC_PARAMS = pltpu.CompilerParams(use_tc_tiling_on_sc=False)


def _widx():
    return lax.axis_index("s") * 2 + lax.axis_index("c")


def _sc_gather(x2a, x2b, src, ce):
    per_w = ce // NW
    n_chunks = per_w // CHUNK
    mesh = plsc.VectorSubcoreMesh(core_axis_name="c", subcore_axis_name="s")

    @functools.partial(
        pl.kernel,
        mesh=mesh,
        out_type=(jax.ShapeDtypeStruct((ce, DA), jnp.float32),
                  jax.ShapeDtypeStruct((ce, DB), jnp.float32)),
        scratch_types=[
            pltpu.VMEM((per_w,), jnp.int32),
            pltpu.VMEM((2, CHUNK, DA), jnp.float32),
            pltpu.VMEM((2, CHUNK, DB), jnp.float32),
            pltpu.SemaphoreType.DMA,
            pltpu.SemaphoreType.DMA,
        ],
        compiler_params=_SC_PARAMS,
    )
    def k(xa_hbm, xb_hbm, src_hbm, outa_hbm, outb_hbm,
          idx_all, rowsA, rowsB, gsem, wsem):
        base = _widx() * per_w
        pltpu.sync_copy(src_hbm.at[pl.ds(base, per_w)], idx_all)

        def gstart(c, b):
            isl = idx_all.at[pl.ds(c * CHUNK, CHUNK)]
            pltpu.async_copy(xa_hbm.at[isl], rowsA.at[b], gsem)
            pltpu.async_copy(xb_hbm.at[isl], rowsB.at[b], gsem)

        def gwait(b):
            isl = idx_all.at[pl.ds(0, CHUNK)]
            pltpu.make_async_copy(xa_hbm.at[isl], rowsA.at[b], gsem).wait()
            pltpu.make_async_copy(xb_hbm.at[isl], rowsB.at[b], gsem).wait()

        def wstart(c, b):
            off = base + c * CHUNK
            pltpu.async_copy(rowsA.at[b], outa_hbm.at[pl.ds(off, CHUNK)], wsem)
            pltpu.async_copy(rowsB.at[b], outb_hbm.at[pl.ds(off, CHUNK)], wsem)

        def wwait(b):
            off = base
            pltpu.make_async_copy(
                rowsA.at[b], outa_hbm.at[pl.ds(off, CHUNK)], wsem).wait()
            pltpu.make_async_copy(
                rowsB.at[b], outb_hbm.at[pl.ds(off, CHUNK)], wsem).wait()

        gstart(0, 0)

        @pl.loop(0, n_chunks - 1, step=2)
        def _pair(j0):
            gwait(0)

            @pl.when(j0 > 0)
            def _():
                wwait(1)
            gstart(j0 + 1, 1)
            wstart(j0, 0)
            gwait(1)
            wwait(0)
            gstart(j0 + 2, 0)
            wstart(j0 + 1, 1)

        gwait(0)
        wwait(1)
        wstart(n_chunks - 1, 0)
        wwait(0)

    return k(x2a, x2b, src)


def _sc_scatter(msgA, msgB, dst, ce):
    per_w = ce // NW
    n_chunks = per_w // CHUNK
    mesh = plsc.VectorSubcoreMesh(core_axis_name="c", subcore_axis_name="s")

    @functools.partial(
        pl.kernel,
        mesh=mesh,
        out_type=(jax.ShapeDtypeStruct((N_NODES, DA), jnp.float32),
                  jax.ShapeDtypeStruct((N_NODES, DA), jnp.float32),
                  jax.ShapeDtypeStruct((N_NODES, DB), jnp.float32),
                  jax.ShapeDtypeStruct((N_NODES, DB), jnp.float32)),
        scratch_types=[
            pltpu.VMEM((2, CHUNK), jnp.int32),
            pltpu.VMEM((2, CHUNK, DA), jnp.float32),
            pltpu.VMEM((2, CHUNK, DB), jnp.float32),
            pltpu.VMEM((WB, DA), jnp.float32),
            pltpu.VMEM((WB, DB), jnp.float32),
            pltpu.VMEM_SHARED((N_NODES, DA), jnp.float32),
            pltpu.VMEM_SHARED((N_NODES, DB), jnp.float32),
            pltpu.SemaphoreType.DMA,
        ],
        compiler_params=_SC_PARAMS,
    )
    def k(ma_hbm, mb_hbm, dst_hbm, pa0_hbm, pa1_hbm, pb0_hbm, pb1_hbm,
          idx2, rowsA, rowsB, zwbA, zwbB, accA, accB, sem):
        cid = lax.axis_index("c")
        sid = lax.axis_index("s")
        base = _widx() * per_w

        zeros16 = jnp.zeros((16,), jnp.float32)

        def zrowA(i, carry):
            def zcol(j, c2):
                zwbA[i, pl.ds(j * 16, 16)] = zeros16
                return c2
            return lax.fori_loop(0, DA // 16, zcol, carry)

        def zrowB(i, carry):
            zwbB[i, :] = zeros16
            return carry

        lax.fori_loop(0, WB, zrowA, 0)
        lax.fori_loop(0, WB, zrowB, 0)

        def zcp(kk, carry):
            r0 = sid * ROWS_PER_TILE + kk * WB
            pltpu.sync_copy(zwbA, accA.at[pl.ds(r0, WB)])
            pltpu.sync_copy(zwbB, accB.at[pl.ds(r0, WB)])
            return carry

        lax.fori_loop(0, N_WB, zcp, 0)
        plsc.subcore_barrier()

        def lstart(c, b):
            off = base + c * CHUNK
            pltpu.async_copy(dst_hbm.at[pl.ds(off, CHUNK)], idx2.at[b], sem)
            pltpu.async_copy(ma_hbm.at[pl.ds(off, CHUNK)], rowsA.at[b], sem)
            pltpu.async_copy(mb_hbm.at[pl.ds(off, CHUNK)], rowsB.at[b], sem)

        def lwait(b):
            off = base
            pltpu.make_async_copy(
                dst_hbm.at[pl.ds(off, CHUNK)], idx2.at[b], sem).wait()
            pltpu.make_async_copy(
                ma_hbm.at[pl.ds(off, CHUNK)], rowsA.at[b], sem).wait()
            pltpu.make_async_copy(
                mb_hbm.at[pl.ds(off, CHUNK)], rowsB.at[b], sem).wait()

        def sadd(b):
            idxr = idx2.at[b]
            pltpu.sync_copy(rowsA.at[b], accA.at[idxr], add=True)
            pltpu.sync_copy(rowsB.at[b], accB.at[idxr], add=True)

        lstart(0, 0)

        @pl.loop(0, n_chunks - 1, step=2)
        def _pair(j0):
            lwait(0)
            lstart(j0 + 1, 1)
            sadd(0)
            lwait(1)
            lstart(j0 + 2, 0)
            sadd(1)

        lwait(0)
        sadd(0)
        plsc.subcore_barrier()

        pa_hbm = [pa0_hbm, pa1_hbm]
        pb_hbm = [pb0_hbm, pb1_hbm]

        def wb(kk, carry):
            r0 = sid * ROWS_PER_TILE + kk * WB
            pltpu.sync_copy(accA.at[pl.ds(r0, WB)], zwbA)
            pltpu.sync_copy(accB.at[pl.ds(r0, WB)], zwbB)
            for c in range(2):
                @pl.when(cid == c)
                def _():
                    pltpu.sync_copy(zwbA, pa_hbm[c].at[pl.ds(r0, WB)])
                    pltpu.sync_copy(zwbB, pb_hbm[c].at[pl.ds(r0, WB)])
            return carry

        lax.fori_loop(0, N_WB, wb, 0)

    return k(msgA, msgB, dst)


def _tc_body(evT_ref, xsa_ref, xsb_ref, w_ref, w1t_ref, b1_ref, w2t_ref,
             b2_ref, w3t_ref, b3_ref, outa_ref, outb_ref):
    f32 = jnp.float32
    ev = evT_ref[...]
    r2 = jnp.sum(ev * ev, axis=0, keepdims=True)
    r = jnp.sqrt(r2)
    inv = 1.0 / (r + 1e-8)
    vx = ev[0:1] * inv
    vy = ev[1:2] * inv
    vz = ev[2:3] * inv
    c0 = 0.28209479177387814
    c1 = 0.4886025119029199
    c2a = 1.0925484305920792
    c2b = 0.31539156525252005
    c2c = 0.5462742152960396
    sh9 = jnp.concatenate([
        c0 * jnp.ones_like(vx),
        c1 * vy,
        c1 * vz,
        c1 * vx,
        c2a * vx * vy,
        c2a * vy * vz,
        c2b * (3.0 * vz * vz - 1.0),
        c2a * vx * vz,
        c2c * (vx * vx - vy * vy),
    ], axis=0)

    centers = (CUTOFF / (N_RBF - 1)) * lax.broadcasted_iota(
        jnp.int32, (N_RBF, 1), 0).astype(f32)
    w = w_ref[...]
    rbf = jnp.exp(-((r - centers) ** 2) / (2.0 * w * w))
    cut = 0.5 * (1.0 + jnp.cos(jnp.pi * r / CUTOFF)) * (r < CUTOFF).astype(f32)

    def mm(a, b):
        return jax.lax.dot_general(a, b, (((1,), (0,)), ((), ())),
                                   preferred_element_type=f32)

    def silu(h):
        return h / (1.0 + jnp.exp(-h))

    h1 = silu(mm(w1t_ref[...], rbf) + b1_ref[...])
    h2 = silu(mm(w2t_ref[...], h1) + b2_ref[...])
    rwT = mm(w3t_ref[...], h2) + b3_ref[...]

    xsT = jnp.concatenate([xsa_ref[...].T, xsb_ref[...].T], axis=0)
    sh9c = sh9 * cut
    shtile = jnp.concatenate([sh9c] * IN_F, axis=0)
    m = xsT * shtile

    parts = []
    for h in range(N_HARM):
        acc = None
        for i in range(IN_F):
            mrow = m[i * N_HARM + h:i * N_HARM + h + 1]
            term = rwT[i * OUT_F:(i + 1) * OUT_F] * mrow
            acc = term if acc is None else acc + term
        parts.append(acc)
    msg = jnp.concatenate(parts, axis=0)
    outa_ref[...] = msg[0:DA].T
    outb_ref[...] = msg[DA:D].T


def _tc_compute(evT, xsA, xsB, widths, w1t, b1c, w2t, b2c, w3tp, b3cp, ce):
    nb = ce // EB
    return pl.pallas_call(
        _tc_body,
        grid=(nb,),
        in_specs=[
            pl.BlockSpec((3, EB), lambda i: (0, i)),
            pl.BlockSpec((EB, DA), lambda i: (i, 0)),
            pl.BlockSpec((EB, DB), lambda i: (i, 0)),
            pl.BlockSpec((N_RBF, 1), lambda i: (0, 0)),
            pl.BlockSpec((64, N_RBF), lambda i: (0, 0)),
            pl.BlockSpec((64, 1), lambda i: (0, 0)),
            pl.BlockSpec((64, 64), lambda i: (0, 0)),
            pl.BlockSpec((64, 1), lambda i: (0, 0)),
            pl.BlockSpec((256, 64), lambda i: (0, 0)),
            pl.BlockSpec((256, 1), lambda i: (0, 0)),
        ],
        out_specs=(pl.BlockSpec((EB, DA), lambda i: (i, 0)),
                   pl.BlockSpec((EB, DB), lambda i: (i, 0))),
        out_shape=(jax.ShapeDtypeStruct((ce, DA), jnp.float32),
                   jax.ShapeDtypeStruct((ce, DB), jnp.float32)),
    )(evT, xsA, xsB, widths, w1t, b1c, w2t, b2c, w3tp, b3cp)


def _combine_body(*refs):
    n = (len(refs) - 2) // 2
    pa = refs[:n]
    pb = refs[n:2 * n]
    oa_ref, ob_ref = refs[-2], refs[-1]
    accA = None
    accB = None
    for ra, rb in zip(pa, pb):
        accA = ra[...] if accA is None else accA + ra[...]
        accB = rb[...] if accB is None else accB + rb[...]
    oa_ref[...] = accA
    ob_ref[...] = accB


def _tc_combine(pAs, pBs):
    nb = 5
    rb = N_NODES // nb
    return pl.pallas_call(
        _combine_body,
        grid=(nb,),
        in_specs=([pl.BlockSpec((rb, DA), lambda i: (i, 0)) for _ in pAs]
                  + [pl.BlockSpec((rb, DB), lambda i: (i, 0)) for _ in pBs]),
        out_specs=(pl.BlockSpec((rb, DA), lambda i: (i, 0)),
                   pl.BlockSpec((rb, DB), lambda i: (i, 0))),
        out_shape=(jax.ShapeDtypeStruct((N_NODES, DA), jnp.float32),
                   jax.ShapeDtypeStruct((N_NODES, DB), jnp.float32)),
    )(*pAs, *pBs)


def kernel(x, edge_index, edge_vec, widths, W1, b1, W2, b2, W3, b3):
    x2 = x.reshape(N_NODES, D)
    x2a = x2[:, :DA]
    x2b = x2[:, DA:]
    src = edge_index[0]
    dst = edge_index[1]
    evT = edge_vec.T

    w1t = W1.T
    w2t = W2.T
    w3p = W3.reshape(64, OUT_F, IN_F).transpose(0, 2, 1).reshape(64, OUT_F * IN_F)
    w3tp = w3p.T
    b3p = b3.reshape(OUT_F, IN_F).T.reshape(OUT_F * IN_F, 1)
    b1c = b1.reshape(64, 1)
    b2c = b2.reshape(64, 1)
    wc = widths.reshape(N_RBF, 1)

    pAs, pBs = [], []
    lo = 0
    for ce in SCHED:
        hi = lo + ce
        xsA, xsB = _sc_gather(x2a, x2b, src[lo:hi], ce)
        msgA, msgB = _tc_compute(evT[:, lo:hi], xsA, xsB, wc, w1t, b1c,
                                 w2t, b2c, w3tp, b3p, ce)
        pa0, pa1, pb0, pb1 = _sc_scatter(msgA, msgB, dst[lo:hi], ce)
        pAs += [pa0, pa1]
        pBs += [pb0, pb1]
        lo = hi

    outA, outB = _tc_combine(pAs, pBs)
    a = outA.reshape(N_NODES, 8, OUT_F)
    b = outB.reshape(N_NODES, 1, OUT_F)
    return jnp.concatenate([a, b], axis=1).transpose(0, 2, 1)


# speedup vs baseline: 1.2281x; 1.0471x over previous
import functools

import jax
import jax.numpy as jnp
from jax import lax
from jax.experimental import pallas as pl
from jax.experimental.pallas import tpu as pltpu
from jax.experimental.pallas import tpu_sc as plsc

N_NODES = 10000
N_EDGES = 320000
IN_F = 16
OUT_F = 16
N_RBF = 16
CUTOFF = 5.0
N_HARM = 9
D = IN_F * N_HARM
DA = 128
DB = D - DA

SCHED = [12800, 89600, 89600, 89600, 38400]
NW = 32
CHUNK = 80

ROWS_PER_TILE = N_NODES // 16
WB = 25
N_WB = ROWS_PER_TILE // WB

EB = 2560

_SC_PARAMS = pltpu.CompilerParams(use_tc_tiling_on_sc=False)


def _widx():
    return lax.axis_index("s") * 2 + lax.axis_index("c")


def _sc_gather(x2a, x2b, src, ce):
    per_w = ce // NW
    n_chunks = per_w // CHUNK
    mesh = plsc.VectorSubcoreMesh(core_axis_name="c", subcore_axis_name="s")

    @functools.partial(
        pl.kernel,
        mesh=mesh,
        out_type=(jax.ShapeDtypeStruct((ce, DA), jnp.float32),
                  jax.ShapeDtypeStruct((ce, DB), jnp.float32)),
        scratch_types=[
            pltpu.VMEM((per_w,), jnp.int32),
            pltpu.VMEM((2, CHUNK, DA), jnp.float32),
            pltpu.VMEM((2, CHUNK, DB), jnp.float32),
            pltpu.SemaphoreType.DMA,
            pltpu.SemaphoreType.DMA,
        ],
        compiler_params=_SC_PARAMS,
    )
    def k(xa_hbm, xb_hbm, src_hbm, outa_hbm, outb_hbm,
          idx_all, rowsA, rowsB, gsem, wsem):
        base = _widx() * per_w
        pltpu.sync_copy(src_hbm.at[pl.ds(base, per_w)], idx_all)

        def gstart(c, b):
            isl = idx_all.at[pl.ds(c * CHUNK, CHUNK)]
            pltpu.async_copy(xa_hbm.at[isl], rowsA.at[b], gsem)
            pltpu.async_copy(xb_hbm.at[isl], rowsB.at[b], gsem)

        def gwait(b):
            isl = idx_all.at[pl.ds(0, CHUNK)]
            pltpu.make_async_copy(xa_hbm.at[isl], rowsA.at[b], gsem).wait()
            pltpu.make_async_copy(xb_hbm.at[isl], rowsB.at[b], gsem).wait()

        def wstart(c, b):
            off = base + c * CHUNK
            pltpu.async_copy(rowsA.at[b], outa_hbm.at[pl.ds(off, CHUNK)], wsem)
            pltpu.async_copy(rowsB.at[b], outb_hbm.at[pl.ds(off, CHUNK)], wsem)

        def wwait(b):
            off = base
            pltpu.make_async_copy(
                rowsA.at[b], outa_hbm.at[pl.ds(off, CHUNK)], wsem).wait()
            pltpu.make_async_copy(
                rowsB.at[b], outb_hbm.at[pl.ds(off, CHUNK)], wsem).wait()

        gstart(0, 0)

        @pl.loop(0, n_chunks - 1, step=2)
        def _pair(j0):
            gwait(0)

            @pl.when(j0 > 0)
            def _():
                wwait(1)
            gstart(j0 + 1, 1)
            wstart(j0, 0)
            gwait(1)
            wwait(0)
            gstart(j0 + 2, 0)
            wstart(j0 + 1, 1)

        gwait(0)
        wwait(1)
        wstart(n_chunks - 1, 0)
        wwait(0)

    return k(x2a, x2b, src)


def _sc_scatter(msgA, msgB, dst, ce):
    per_w = ce // NW
    n_chunks = per_w // CHUNK
    mesh = plsc.VectorSubcoreMesh(core_axis_name="c", subcore_axis_name="s")

    @functools.partial(
        pl.kernel,
        mesh=mesh,
        out_type=(jax.ShapeDtypeStruct((N_NODES, DA), jnp.float32),
                  jax.ShapeDtypeStruct((N_NODES, DA), jnp.float32),
                  jax.ShapeDtypeStruct((N_NODES, DB), jnp.float32),
                  jax.ShapeDtypeStruct((N_NODES, DB), jnp.float32)),
        scratch_types=[
            pltpu.VMEM((2, CHUNK), jnp.int32),
            pltpu.VMEM((2, CHUNK, DA), jnp.float32),
            pltpu.VMEM((2, CHUNK, DB), jnp.float32),
            pltpu.VMEM((WB, DA), jnp.float32),
            pltpu.VMEM((WB, DB), jnp.float32),
            pltpu.VMEM_SHARED((N_NODES, DA), jnp.float32),
            pltpu.VMEM_SHARED((N_NODES, DB), jnp.float32),
            pltpu.SemaphoreType.DMA,
        ],
        compiler_params=_SC_PARAMS,
    )
    def k(ma_hbm, mb_hbm, dst_hbm, pa0_hbm, pa1_hbm, pb0_hbm, pb1_hbm,
          idx2, rowsA, rowsB, zwbA, zwbB, accA, accB, sem):
        cid = lax.axis_index("c")
        sid = lax.axis_index("s")
        base = _widx() * per_w

        zeros16 = jnp.zeros((16,), jnp.float32)

        def zrowA(i, carry):
            def zcol(j, c2):
                zwbA[i, pl.ds(j * 16, 16)] = zeros16
                return c2
            return lax.fori_loop(0, DA // 16, zcol, carry)

        def zrowB(i, carry):
            zwbB[i, :] = zeros16
            return carry

        lax.fori_loop(0, WB, zrowA, 0)
        lax.fori_loop(0, WB, zrowB, 0)

        def zcp(kk, carry):
            r0 = sid * ROWS_PER_TILE + kk * WB
            pltpu.sync_copy(zwbA, accA.at[pl.ds(r0, WB)])
            pltpu.sync_copy(zwbB, accB.at[pl.ds(r0, WB)])
            return carry

        lax.fori_loop(0, N_WB, zcp, 0)
        plsc.subcore_barrier()

        def lstart(c, b):
            off = base + c * CHUNK
            pltpu.async_copy(dst_hbm.at[pl.ds(off, CHUNK)], idx2.at[b], sem)
            pltpu.async_copy(ma_hbm.at[pl.ds(off, CHUNK)], rowsA.at[b], sem)
            pltpu.async_copy(mb_hbm.at[pl.ds(off, CHUNK)], rowsB.at[b], sem)

        def lwait(b):
            off = base
            pltpu.make_async_copy(
                dst_hbm.at[pl.ds(off, CHUNK)], idx2.at[b], sem).wait()
            pltpu.make_async_copy(
                ma_hbm.at[pl.ds(off, CHUNK)], rowsA.at[b], sem).wait()
            pltpu.make_async_copy(
                mb_hbm.at[pl.ds(off, CHUNK)], rowsB.at[b], sem).wait()

        def sadd(b):
            idxr = idx2.at[b]
            pltpu.sync_copy(rowsA.at[b], accA.at[idxr], add=True)
            pltpu.sync_copy(rowsB.at[b], accB.at[idxr], add=True)

        lstart(0, 0)

        @pl.loop(0, n_chunks - 1, step=2)
        def _pair(j0):
            lwait(0)
            lstart(j0 + 1, 1)
            sadd(0)
            lwait(1)
            lstart(j0 + 2, 0)
            sadd(1)

        lwait(0)
        sadd(0)
        plsc.subcore_barrier()

        pa_hbm = [pa0_hbm, pa1_hbm]
        pb_hbm = [pb0_hbm, pb1_hbm]

        def wb(kk, carry):
            r0 = sid * ROWS_PER_TILE + kk * WB
            pltpu.sync_copy(accA.at[pl.ds(r0, WB)], zwbA)
            pltpu.sync_copy(accB.at[pl.ds(r0, WB)], zwbB)
            for c in range(2):
                @pl.when(cid == c)
                def _():
                    pltpu.sync_copy(zwbA, pa_hbm[c].at[pl.ds(r0, WB)])
                    pltpu.sync_copy(zwbB, pb_hbm[c].at[pl.ds(r0, WB)])
            return carry

        lax.fori_loop(0, N_WB, wb, 0)

    return k(msgA, msgB, dst)


def _tc_body(evT_ref, xsa_ref, xsb_ref, w_ref, w1t_ref, b1_ref, w2t_ref,
             b2_ref, w3t_ref, b3_ref, outa_ref, outb_ref):
    f32 = jnp.float32
    ev = evT_ref[...]
    r2 = jnp.sum(ev * ev, axis=0, keepdims=True)
    r = jnp.sqrt(r2)
    inv = 1.0 / (r + 1e-8)
    vx = ev[0:1] * inv
    vy = ev[1:2] * inv
    vz = ev[2:3] * inv
    c0 = 0.28209479177387814
    c1 = 0.4886025119029199
    c2a = 1.0925484305920792
    c2b = 0.31539156525252005
    c2c = 0.5462742152960396
    sh9 = jnp.concatenate([
        c0 * jnp.ones_like(vx),
        c1 * vy,
        c1 * vz,
        c1 * vx,
        c2a * vx * vy,
        c2a * vy * vz,
        c2b * (3.0 * vz * vz - 1.0),
        c2a * vx * vz,
        c2c * (vx * vx - vy * vy),
    ], axis=0)

    centers = (CUTOFF / (N_RBF - 1)) * lax.broadcasted_iota(
        jnp.int32, (N_RBF, 1), 0).astype(f32)
    w = w_ref[...]
    rbf = jnp.exp(-((r - centers) ** 2) / (2.0 * w * w))
    cut = 0.5 * (1.0 + jnp.cos(jnp.pi * r / CUTOFF)) * (r < CUTOFF).astype(f32)

    def mm(a, b):
        return jax.lax.dot_general(a, b, (((1,), (0,)), ((), ())),
                                   preferred_element_type=f32)

    def silu(h):
        return h / (1.0 + jnp.exp(-h))

    h1 = silu(mm(w1t_ref[...], rbf) + b1_ref[...])
    h2 = silu(mm(w2t_ref[...], h1) + b2_ref[...])
    rwT = mm(w3t_ref[...], h2) + b3_ref[...]

    xsT = jnp.concatenate([xsa_ref[...].T, xsb_ref[...].T], axis=0)
    sh9c = sh9 * cut
    shtile = jnp.concatenate([sh9c] * IN_F, axis=0)
    m = xsT * shtile

    parts = []
    for h in range(N_HARM):
        acc = None
        for i in range(IN_F):
            mrow = m[i * N_HARM + h:i * N_HARM + h + 1]
            term = rwT[i * OUT_F:(i + 1) * OUT_F] * mrow
            acc = term if acc is None else acc + term
        parts.append(acc)
    msg = jnp.concatenate(parts, axis=0)
    outa_ref[...] = msg[0:DA].T
    outb_ref[...] = msg[DA:D].T


def _tc_compute(evT, xsA, xsB, widths, w1t, b1c, w2t, b2c, w3tp, b3cp, ce):
    nb = ce // EB
    return pl.pallas_call(
        _tc_body,
        grid=(nb,),
        in_specs=[
            pl.BlockSpec((3, EB), lambda i: (0, i)),
            pl.BlockSpec((EB, DA), lambda i: (i, 0)),
            pl.BlockSpec((EB, DB), lambda i: (i, 0)),
            pl.BlockSpec((N_RBF, 1), lambda i: (0, 0)),
            pl.BlockSpec((64, N_RBF), lambda i: (0, 0)),
            pl.BlockSpec((64, 1), lambda i: (0, 0)),
            pl.BlockSpec((64, 64), lambda i: (0, 0)),
            pl.BlockSpec((64, 1), lambda i: (0, 0)),
            pl.BlockSpec((256, 64), lambda i: (0, 0)),
            pl.BlockSpec((256, 1), lambda i: (0, 0)),
        ],
        out_specs=(pl.BlockSpec((EB, DA), lambda i: (i, 0)),
                   pl.BlockSpec((EB, DB), lambda i: (i, 0))),
        out_shape=(jax.ShapeDtypeStruct((ce, DA), jnp.float32),
                   jax.ShapeDtypeStruct((ce, DB), jnp.float32)),
    )(evT, xsA, xsB, widths, w1t, b1c, w2t, b2c, w3tp, b3cp)


def _combine_body(*refs):
    n = (len(refs) - 2) // 2
    pa = refs[:n]
    pb = refs[n:2 * n]
    oa_ref, ob_ref = refs[-2], refs[-1]
    accA = None
    accB = None
    for ra, rb in zip(pa, pb):
        accA = ra[...] if accA is None else accA + ra[...]
        accB = rb[...] if accB is None else accB + rb[...]
    oa_ref[...] = accA
    ob_ref[...] = accB


def _tc_combine(pAs, pBs):
    nb = 5
    rb = N_NODES // nb
    return pl.pallas_call(
        _combine_body,
        grid=(nb,),
        in_specs=([pl.BlockSpec((rb, DA), lambda i: (i, 0)) for _ in pAs]
                  + [pl.BlockSpec((rb, DB), lambda i: (i, 0)) for _ in pBs]),
        out_specs=(pl.BlockSpec((rb, DA), lambda i: (i, 0)),
                   pl.BlockSpec((rb, DB), lambda i: (i, 0))),
        out_shape=(jax.ShapeDtypeStruct((N_NODES, DA), jnp.float32),
                   jax.ShapeDtypeStruct((N_NODES, DB), jnp.float32)),
    )(*pAs, *pBs)


def kernel(x, edge_index, edge_vec, widths, W1, b1, W2, b2, W3, b3):
    x2 = x.reshape(N_NODES, D)
    x2a = x2[:, :DA]
    x2b = x2[:, DA:]
    src = edge_index[0]
    dst = edge_index[1]
    evT = edge_vec.T

    w1t = W1.T
    w2t = W2.T
    w3p = W3.reshape(64, OUT_F, IN_F).transpose(0, 2, 1).reshape(64, OUT_F * IN_F)
    w3tp = w3p.T
    b3p = b3.reshape(OUT_F, IN_F).T.reshape(OUT_F * IN_F, 1)
    b1c = b1.reshape(64, 1)
    b2c = b2.reshape(64, 1)
    wc = widths.reshape(N_RBF, 1)

    pAs, pBs = [], []
    lo = 0
    for ce in SCHED:
        hi = lo + ce
        xsA, xsB = _sc_gather(x2a, x2b, src[lo:hi], ce)
        msgA, msgB = _tc_compute(evT[:, lo:hi], xsA, xsB, wc, w1t, b1c,
                                 w2t, b2c, w3tp, b3p, ce)
        pa0, pa1, pb0, pb1 = _sc_scatter(msgA, msgB, dst[lo:hi], ce)
        pAs += [pa0, pa1]
        pBs += [pb0, pb1]
        lo = hi

    outA, outB = _tc_combine(pAs, pBs)
    a = outA.reshape(N_NODES, 8, OUT_F)
    b = outB.reshape(N_NODES, 1, OUT_F)
    return jnp.concatenate([a, b], axis=1).transpose(0, 2, 1)
